# Initial kernel scaffold; baseline (speedup 1.0000x reference)
import jax
import jax.numpy as jnp
from jax.experimental import pallas as pl


def kernel(outputs, targets, mask):
    raise NotImplementedError("write your pallas kernel here")


# speedup vs baseline: 9.1787x; 9.1787x over previous
import jax
import jax.numpy as jnp
from jax import lax
from jax.experimental import pallas as pl
from jax.experimental.pallas import tpu as pltpu

_K = 16
_MAXP = 512
_FMAX = 3.4e38


def _bottom_k_rowsum(s, k):
    rows = s.shape[0]
    acc = jnp.zeros((rows, 1), jnp.float32)
    krem = jnp.full((rows, 1), jnp.float32(k))
    for _ in range(k):
        m = jnp.min(s, axis=1, keepdims=True)
        eq = s == m
        c = jnp.sum(eq.astype(jnp.float32), axis=1, keepdims=True)
        take = jnp.minimum(c, krem)
        acc = acc + jnp.where(take > 0, m * take, 0.0)
        krem = krem - take
        s = jnp.where(eq, _FMAX, s)
    return acc


def _gj_inverse(a, n):
    ri = lax.broadcasted_iota(jnp.int32, (n, 1), 0)
    ci = lax.broadcasted_iota(jnp.int32, (1, 2 * n), 1)
    eye = (ci[:, :n] == ri).astype(jnp.float32)
    aug = jnp.concatenate([a, eye], axis=1)
    for kk in range(n):
        piv = aug[kk:kk + 1, kk:kk + 1]
        rowk = aug[kk:kk + 1, :] / piv
        colk = aug[:, kk:kk + 1]
        aug = jnp.where(ri == kk, rowk, aug - colk * rowk)
    return aug[:, n:]


def _loss_body(t_ref, mk_ref, p_ref, pt_ref, oi_ref, oo_ref):
    t = t_ref[...]
    mk = mk_ref[...]
    p = p_ref[...]
    pt = pt_ref[...]

    bm = t.shape[0]
    bn = p.shape[0]
    d = t.shape[1]
    n = pt.shape[1]
    b = bn // n
    m = bm // b

    count = jnp.sum(mk)
    mean = jnp.sum(t * mk, axis=0, keepdims=True) / count
    am = (t - mean) * mk
    cov_rows = [
        jnp.sum(am[:, i:i + 1] * am, axis=0, keepdims=True) for i in range(d)
    ]
    cov = jnp.concatenate(cov_rows, axis=0) / (count - 1.0)
    cinv = _gj_inverse(cov, d)

    a = t - mean
    ga = jnp.dot(a, cinv, preferred_element_type=jnp.float32)
    alpha = jnp.sum(ga * a, axis=1, keepdims=True)
    c = p + mean
    gc = jnp.dot(c, cinv, preferred_element_type=jnp.float32)
    gamma = jnp.sum(gc * c, axis=1, keepdims=True)

    intra_acc = jnp.float32(0.0)
    outer_acc = jnp.float32(0.0)
    for bi in range(b):
        ptb = pt[bi * d:(bi + 1) * d, :]
        cp = jnp.dot(cinv, ptb, preferred_element_type=jnp.float32)
        beta = jnp.sum(cp * ptb, axis=0, keepdims=True)

        gab = ga[bi * m:(bi + 1) * m, :]
        si = alpha[bi * m:(bi + 1) * m, :] + beta - 2.0 * jnp.dot(
            gab, ptb, preferred_element_type=jnp.float32)
        bs_i = _bottom_k_rowsum(si, _K)
        mb = mk[bi * m:(bi + 1) * m, :]
        intra_acc = intra_acc + jnp.sum(bs_i * mb)

        gcb = gc[bi * n:(bi + 1) * n, :]
        so = gamma[bi * n:(bi + 1) * n, :] + beta - 2.0 * jnp.dot(
            gcb, ptb, preferred_element_type=jnp.float32)
        bs_o = _bottom_k_rowsum(so, _K)
        outer_acc = outer_acc + jnp.sum(bs_o)

    oi_ref[0, 0] = intra_acc / count
    oo_ref[0, 0] = outer_acc / jnp.float32(bn * _K)


def kernel(outputs, targets, mask):
    bsz = targets.shape[0]
    msz = targets.shape[1]
    d = targets.shape[2]
    y_pred = outputs[:, :_MAXP]
    nsz = y_pred.shape[1]

    t_flat = targets.reshape(bsz * msz, d)
    mk = mask.reshape(bsz * msz, 1).astype(jnp.float32)
    p_flat = y_pred.reshape(bsz * nsz, d)
    p_t = y_pred.transpose(0, 2, 1).reshape(bsz * d, nsz)

    intra, outer = pl.pallas_call(
        _loss_body,
        out_shape=[
            jax.ShapeDtypeStruct((1, 1), jnp.float32),
            jax.ShapeDtypeStruct((1, 1), jnp.float32),
        ],
        out_specs=[
            pl.BlockSpec(memory_space=pltpu.SMEM),
            pl.BlockSpec(memory_space=pltpu.SMEM),
        ],
    )(t_flat, mk, p_flat, p_t)

    intra_loss = intra[0, 0]
    outer_loss = outer[0, 0]
    return (intra_loss, intra_loss, outer_loss)


# speedup vs baseline: 9.6928x; 1.0560x over previous
import jax
import jax.numpy as jnp
from jax import lax
from jax.experimental import pallas as pl
from jax.experimental.pallas import tpu as pltpu

_K = 16
_MAXP = 512
_FMAX = 3.4e38


def _bottom_k_colsum(s, k):
    cols = s.shape[1]
    acc = jnp.zeros((1, cols), jnp.float32)
    krem = jnp.full((1, cols), jnp.float32(k))
    for _ in range(k):
        m = jnp.min(s, axis=0, keepdims=True)
        eq = s == m
        c = jnp.sum(eq.astype(jnp.float32), axis=0, keepdims=True)
        take = jnp.minimum(c, krem)
        acc = acc + jnp.where(take > 0, m * take, 0.0)
        krem = krem - take
        s = jnp.where(eq, _FMAX, s)
    return acc


def _gj_inverse(a, n):
    ri = lax.broadcasted_iota(jnp.int32, (n, 1), 0)
    for kk in range(n):
        piv = a[kk:kk + 1, kk:kk + 1]
        rowk = a[kk:kk + 1, :] / piv
        colk = a[:, kk:kk + 1]
        a = jnp.where(ri == kk, rowk, a - colk * rowk)
    return a


def _loss_body(t_ref, mk_ref, tt_ref, p_ref, pt_ref, mrow_ref,
               oi_ref, oo_ref):
    t = t_ref[...]
    mk = mk_ref[...]
    tt = tt_ref[...]
    p = p_ref[...]
    pt = pt_ref[...]
    mrow = mrow_ref[...]

    bm = t.shape[0]
    bn = p.shape[0]
    d = t.shape[1]
    n = pt.shape[1]
    b = bn // n
    m = bm // b

    count = jnp.sum(mk)
    mean = jnp.sum(t * mk, axis=0, keepdims=True) / count
    am = (t - mean) * mk
    cov_rows = [
        jnp.sum(am[:, i:i + 1] * am, axis=0, keepdims=True) for i in range(d)
    ]
    cov = jnp.concatenate(cov_rows, axis=0) / (count - 1.0)
    ri = lax.broadcasted_iota(jnp.int32, (d, 1), 0)
    ci = lax.broadcasted_iota(jnp.int32, (1, d), 1)
    eye = (ci == ri).astype(jnp.float32)
    cinv = _gj_inverse(jnp.concatenate([cov, eye], axis=1), d)[:, d:]
    meant = jnp.sum(eye * mean, axis=1, keepdims=True)

    gp = jnp.dot(p, cinv, preferred_element_type=jnp.float32)
    beta = jnp.sum(gp * p, axis=1, keepdims=True)

    intra_acc = jnp.float32(0.0)
    outer_acc = jnp.float32(0.0)
    for bi in range(b):
        atb = tt[bi * d:(bi + 1) * d, :] - meant
        cat = jnp.dot(cinv, atb, preferred_element_type=jnp.float32)
        alpha = jnp.sum(atb * cat, axis=0, keepdims=True)
        gpb = gp[bi * n:(bi + 1) * n, :]
        betac = beta[bi * n:(bi + 1) * n, :]

        sit = betac + alpha - 2.0 * jnp.dot(
            gpb, atb, preferred_element_type=jnp.float32)
        acc_i = _bottom_k_colsum(sit, _K)
        intra_acc = intra_acc + jnp.sum(acc_i * mrow[bi:bi + 1, :])

        ctb = pt[bi * d:(bi + 1) * d, :] + meant
        cct = jnp.dot(cinv, ctb, preferred_element_type=jnp.float32)
        gamma = jnp.sum(ctb * cct, axis=0, keepdims=True)
        sot = betac + gamma - 2.0 * jnp.dot(
            gpb, ctb, preferred_element_type=jnp.float32)
        acc_o = _bottom_k_colsum(sot, _K)
        outer_acc = outer_acc + jnp.sum(acc_o)

    oi_ref[0, 0] = intra_acc / count
    oo_ref[0, 0] = outer_acc / jnp.float32(bn * _K)


def kernel(outputs, targets, mask):
    bsz = targets.shape[0]
    msz = targets.shape[1]
    d = targets.shape[2]
    y_pred = outputs[:, :_MAXP]
    nsz = y_pred.shape[1]

    t_flat = targets.reshape(bsz * msz, d)
    t_t = targets.transpose(0, 2, 1).reshape(bsz * d, msz)
    mk = mask.reshape(bsz * msz, 1).astype(jnp.float32)
    mrow = mask.astype(jnp.float32)
    p_flat = y_pred.reshape(bsz * nsz, d)
    p_t = y_pred.transpose(0, 2, 1).reshape(bsz * d, nsz)

    intra, outer = pl.pallas_call(
        _loss_body,
        out_shape=[
            jax.ShapeDtypeStruct((1, 1), jnp.float32),
            jax.ShapeDtypeStruct((1, 1), jnp.float32),
        ],
        out_specs=[
            pl.BlockSpec(memory_space=pltpu.SMEM),
            pl.BlockSpec(memory_space=pltpu.SMEM),
        ],
    )(t_flat, mk, t_t, p_flat, p_t, mrow)

    intra_loss = intra[0, 0]
    outer_loss = outer[0, 0]
    return (intra_loss, intra_loss, outer_loss)


# speedup vs baseline: 11.6188x; 1.1987x over previous
import jax
import jax.numpy as jnp
from jax import lax
from jax.experimental import pallas as pl
from jax.experimental.pallas import tpu as pltpu

_K = 16
_MAXP = 512
_IMAX = 0x7FFFFFFF


def _to_key(s):
    b = lax.bitcast_convert_type(s, jnp.int32)
    key = b ^ (lax.shift_right_arithmetic(b, 31) & _IMAX)
    ridx = lax.broadcasted_iota(jnp.int32, s.shape, 0)
    return (key & -512) | ridx


def _from_key(key):
    b = key ^ (lax.shift_right_arithmetic(key, 31) & _IMAX)
    return lax.bitcast_convert_type(b, jnp.float32)


def _bottom_k_colsum(s, k):
    cols = s.shape[1]
    key = _to_key(s)
    acc = jnp.zeros((1, cols), jnp.float32)
    for _ in range(k):
        mkey = jnp.min(key, axis=0, keepdims=True)
        acc = acc + _from_key(mkey)
        key = jnp.where(key == mkey, _IMAX, key)
    return acc


def _gj_inverse(a, n):
    ri = lax.broadcasted_iota(jnp.int32, (n, 1), 0)
    for kk in range(n):
        piv = a[kk:kk + 1, kk:kk + 1]
        rowk = a[kk:kk + 1, :] / piv
        colk = a[:, kk:kk + 1]
        a = jnp.where(ri == kk, rowk, a - colk * rowk)
    return a


def _loss_body(t_ref, mk_ref, tt_ref, p_ref, pt_ref, mrow_ref,
               oi_ref, oo_ref):
    t = t_ref[...]
    mk = mk_ref[...]
    tt = tt_ref[...]
    p = p_ref[...]
    pt = pt_ref[...]
    mrow = mrow_ref[...]

    bm = t.shape[0]
    bn = p.shape[0]
    d = t.shape[1]
    n = pt.shape[1]
    b = bn // n
    m = bm // b

    count = jnp.sum(mk)
    mean = jnp.sum(t * mk, axis=0, keepdims=True) / count
    am = (t - mean) * mk
    cov_rows = [
        jnp.sum(am[:, i:i + 1] * am, axis=0, keepdims=True) for i in range(d)
    ]
    cov = jnp.concatenate(cov_rows, axis=0) / (count - 1.0)
    ri = lax.broadcasted_iota(jnp.int32, (d, 1), 0)
    ci = lax.broadcasted_iota(jnp.int32, (1, d), 1)
    eye = (ci == ri).astype(jnp.float32)
    cinv = _gj_inverse(jnp.concatenate([cov, eye], axis=1), d)[:, d:]
    meant = jnp.sum(eye * mean, axis=1, keepdims=True)

    gp = jnp.dot(p, cinv, preferred_element_type=jnp.float32)
    beta = jnp.sum(gp * p, axis=1, keepdims=True)
    gpaug = jnp.concatenate([-2.0 * gp, beta], axis=1)

    intra_acc = jnp.float32(0.0)
    outer_acc = jnp.float32(0.0)
    kf = jnp.float32(_K)
    for bi in range(b):
        gpb = gpaug[bi * n:(bi + 1) * n, :]

        atb = tt[bi * d:(bi + 1) * d, :] - meant
        cat = jnp.dot(cinv, atb, preferred_element_type=jnp.float32)
        alpha = jnp.sum(atb * cat, axis=0, keepdims=True)
        ataug = jnp.concatenate(
            [atb, jnp.ones((1, atb.shape[1]), jnp.float32)], axis=0)
        sit = jnp.dot(gpb, ataug, preferred_element_type=jnp.float32)
        acc_i = _bottom_k_colsum(sit, _K) + kf * alpha
        intra_acc = intra_acc + jnp.sum(acc_i * mrow[bi:bi + 1, :])

        ctb = pt[bi * d:(bi + 1) * d, :] + meant
        cct = jnp.dot(cinv, ctb, preferred_element_type=jnp.float32)
        gamma = jnp.sum(ctb * cct, axis=0, keepdims=True)
        ctaug = jnp.concatenate(
            [ctb, jnp.ones((1, ctb.shape[1]), jnp.float32)], axis=0)
        sot = jnp.dot(gpb, ctaug, preferred_element_type=jnp.float32)
        acc_o = _bottom_k_colsum(sot, _K) + kf * gamma
        outer_acc = outer_acc + jnp.sum(acc_o)

    oi_ref[0, 0] = intra_acc / count
    oo_ref[0, 0] = outer_acc / jnp.float32(bn * _K)


def kernel(outputs, targets, mask):
    bsz = targets.shape[0]
    msz = targets.shape[1]
    d = targets.shape[2]
    y_pred = outputs[:, :_MAXP]
    nsz = y_pred.shape[1]

    t_flat = targets.reshape(bsz * msz, d)
    t_t = targets.transpose(0, 2, 1).reshape(bsz * d, msz)
    mk = mask.reshape(bsz * msz, 1).astype(jnp.float32)
    mrow = mask.astype(jnp.float32)
    p_flat = y_pred.reshape(bsz * nsz, d)
    p_t = y_pred.transpose(0, 2, 1).reshape(bsz * d, nsz)

    intra, outer = pl.pallas_call(
        _loss_body,
        out_shape=[
            jax.ShapeDtypeStruct((1, 1), jnp.float32),
            jax.ShapeDtypeStruct((1, 1), jnp.float32),
        ],
        out_specs=[
            pl.BlockSpec(memory_space=pltpu.SMEM),
            pl.BlockSpec(memory_space=pltpu.SMEM),
        ],
    )(t_flat, mk, t_t, p_flat, p_t, mrow)

    intra_loss = intra[0, 0]
    outer_loss = outer[0, 0]
    return (intra_loss, intra_loss, outer_loss)


# speedup vs baseline: 15.2086x; 1.3090x over previous
import jax
import jax.numpy as jnp
from jax import lax
from jax.experimental import pallas as pl
from jax.experimental.pallas import tpu as pltpu

_K = 16
_MAXP = 512
_FMAX = 3.4e38


def _to_key(s):
    b = lax.bitcast_convert_type(s, jnp.int32)
    ridx = lax.broadcasted_iota(jnp.int32, s.shape, 0)
    return lax.bitcast_convert_type((b & -512) | ridx, jnp.float32)


def _bottom_k_colsum(s, k):
    cols = s.shape[1]
    key = _to_key(s)
    acc = jnp.zeros((1, cols), jnp.float32)
    for _ in range(k):
        mkey = jnp.min(key, axis=0, keepdims=True)
        acc = acc + mkey
        key = jnp.where(key == mkey, _FMAX, key)
    return acc


def _gj_inverse(a, n):
    ri = lax.broadcasted_iota(jnp.int32, (n, 1), 0)
    for kk in range(n):
        piv = a[kk:kk + 1, kk:kk + 1]
        rowk = a[kk:kk + 1, :] / piv
        colk = a[:, kk:kk + 1]
        a = jnp.where(ri == kk, rowk, a - colk * rowk)
    return a


def _loss_body(t_ref, mk_ref, tt_ref, p_ref, pt_ref, mrow_ref,
               oi_ref, oo_ref):
    t = t_ref[...]
    mk = mk_ref[...]
    tt = tt_ref[...]
    p = p_ref[...]
    pt = pt_ref[...]
    mrow = mrow_ref[...]

    bm = t.shape[0]
    bn = p.shape[0]
    d = t.shape[1]
    n = pt.shape[1]
    b = bn // n
    m = bm // b

    count = jnp.sum(mk)
    mean = jnp.sum(t * mk, axis=0, keepdims=True) / count
    am = (t - mean) * mk
    cov = lax.dot_general(
        am, am, (((0,), (0,)), ((), ())),
        preferred_element_type=jnp.float32) / (count - 1.0)
    ri = lax.broadcasted_iota(jnp.int32, (d, 1), 0)
    ci = lax.broadcasted_iota(jnp.int32, (1, d), 1)
    eye = (ci == ri).astype(jnp.float32)
    cinv = _gj_inverse(jnp.concatenate([cov, eye], axis=1), d)[:, d:]
    meant = jnp.sum(eye * mean, axis=1, keepdims=True)

    gp = jnp.dot(p, cinv, preferred_element_type=jnp.float32)
    beta = jnp.sum(gp * p, axis=1, keepdims=True)
    gpaug = jnp.concatenate([-2.0 * gp, beta], axis=1)

    intra_acc = jnp.float32(0.0)
    outer_acc = jnp.float32(0.0)
    kf = jnp.float32(_K)
    for bi in range(b):
        gpb = gpaug[bi * n:(bi + 1) * n, :]

        atb = tt[bi * d:(bi + 1) * d, :] - meant
        cat = jnp.dot(cinv, atb, preferred_element_type=jnp.float32)
        alpha = jnp.sum(atb * cat, axis=0, keepdims=True)
        ataug = jnp.concatenate(
            [atb, jnp.ones((1, atb.shape[1]), jnp.float32)], axis=0)
        sit = jnp.dot(gpb, ataug, preferred_element_type=jnp.float32)
        acc_i = _bottom_k_colsum(sit, _K) + kf * alpha
        intra_acc = intra_acc + jnp.sum(acc_i * mrow[bi:bi + 1, :])

        ctb = pt[bi * d:(bi + 1) * d, :] + meant
        cct = jnp.dot(cinv, ctb, preferred_element_type=jnp.float32)
        gamma = jnp.sum(ctb * cct, axis=0, keepdims=True)
        ctaug = jnp.concatenate(
            [ctb, jnp.ones((1, ctb.shape[1]), jnp.float32)], axis=0)
        sot = jnp.dot(gpb, ctaug, preferred_element_type=jnp.float32)
        acc_o = _bottom_k_colsum(sot, _K) + kf * gamma
        outer_acc = outer_acc + jnp.sum(acc_o)

    oi_ref[0, 0] = intra_acc / count
    oo_ref[0, 0] = outer_acc / jnp.float32(bn * _K)


def kernel(outputs, targets, mask):
    bsz = targets.shape[0]
    msz = targets.shape[1]
    d = targets.shape[2]
    y_pred = outputs[:, :_MAXP]
    nsz = y_pred.shape[1]

    t_flat = targets.reshape(bsz * msz, d)
    t_t = targets.transpose(0, 2, 1).reshape(bsz * d, msz)
    mk = mask.reshape(bsz * msz, 1).astype(jnp.float32)
    mrow = mask.astype(jnp.float32)
    p_flat = y_pred.reshape(bsz * nsz, d)
    p_t = y_pred.transpose(0, 2, 1).reshape(bsz * d, nsz)

    intra, outer = pl.pallas_call(
        _loss_body,
        out_shape=[
            jax.ShapeDtypeStruct((1, 1), jnp.float32),
            jax.ShapeDtypeStruct((1, 1), jnp.float32),
        ],
        out_specs=[
            pl.BlockSpec(memory_space=pltpu.SMEM),
            pl.BlockSpec(memory_space=pltpu.SMEM),
        ],
    )(t_flat, mk, t_t, p_flat, p_t, mrow)

    intra_loss = intra[0, 0]
    outer_loss = outer[0, 0]
    return (intra_loss, intra_loss, outer_loss)
